# Initial kernel scaffold; baseline (speedup 1.0000x reference)
import jax
import jax.numpy as jnp
from jax.experimental import pallas as pl


def kernel(attribute_triples, ent_edges, ent_edge_labels, val_edges, val_edge_labels, att_feats, val_feats, ent_feats, W, gcn1_W, gcn1_b, gcn2_W, gcn2_b, gat1_W, gat1_att_src, gat1_att_dst, gat1_bias, gat1re_W, gat1re_att_src, gat1re_att_dst, gat1re_bias):
    raise NotImplementedError("write your pallas kernel here")


# speedup vs baseline: 1.1996x; 1.1996x over previous
import jax
import jax.numpy as jnp
from jax.experimental import pallas as pl


def _mm_bias(x, w, b, block=2000):
    n, k = x.shape
    m = w.shape[1]
    assert n % block == 0

    def body(x_ref, w_ref, b_ref, o_ref):
        o_ref[...] = (
            jnp.dot(x_ref[...], w_ref[...], preferred_element_type=jnp.float32)
            + b_ref[...]
        )

    return pl.pallas_call(
        body,
        grid=(n // block,),
        in_specs=[
            pl.BlockSpec((block, k), lambda i: (i, 0)),
            pl.BlockSpec((k, m), lambda i: (0, 0)),
            pl.BlockSpec((1, m), lambda i: (0, 0)),
        ],
        out_specs=pl.BlockSpec((block, m), lambda i: (i, 0)),
        out_shape=jax.ShapeDtypeStruct((n, m), jnp.float32),
    )(x, w, b.reshape(1, m))


def kernel(attribute_triples, ent_edges, ent_edge_labels, val_edges,
           val_edge_labels, att_feats, val_feats, ent_feats, W, gcn1_W,
           gcn1_b, gcn2_W, gcn2_b, gat1_W, gat1_att_src, gat1_att_dst,
           gat1_bias, gat1re_W, gat1re_att_src, gat1re_att_dst, gat1re_bias):
    val = attribute_triples[:, 1]
    att = attribute_triples[:, 2]
    num_ent = ent_feats.shape[0]
    key_dim = ent_feats.shape[1]
    n = num_ent + val.shape[0]

    att_proj = att_feats @ W[:key_dim]
    val_proj = val_feats @ W[key_dim:]
    vfeat = att_proj[att] + val_proj[val]
    x0 = jnp.concatenate([ent_feats, vfeat], axis=0)

    ar = jnp.arange(n, dtype=val_edges.dtype)
    ve_row = jnp.concatenate([val_edges[:, 0], ar])
    ve_col = jnp.concatenate([val_edges[:, 1], ar])
    deg = jax.ops.segment_sum(jnp.ones(ve_col.shape[0], jnp.float32), ve_col,
                              num_segments=n)
    dinv = jnp.where(deg > 0, deg ** -0.5, 0.0)
    norm = dinv[ve_row] * dinv[ve_col]

    def agg(x):
        return jax.ops.segment_sum(x[ve_row] * norm[:, None], ve_col,
                                   num_segments=n)

    x1 = _mm_bias(agg(x0), gcn1_W, gcn1_b)
    x2 = _mm_bias(agg(x1), gcn2_W, gcn2_b)

    ee_row = jnp.concatenate([ent_edges[:, 0], ar])
    ee_col = jnp.concatenate([ent_edges[:, 1], ar])

    u1 = gat1_W @ gat1_att_src
    v1 = gat1_W @ gat1_att_dst
    u2 = gat1re_W @ gat1re_att_src
    v2 = gat1re_W @ gat1re_att_dst
    proj = x2 @ jnp.stack([u1, v1, u2, v2], axis=1)

    def gat_scalars(s_idx, d_idx):
        al = proj[:, s_idx][ee_row] + proj[:, d_idx][ee_col]
        al = jnp.where(al > 0, al, 0.2 * al)
        m = jax.ops.segment_max(al, ee_col, num_segments=n)
        m = jnp.where(jnp.isfinite(m), m, 0.0)
        ex = jnp.exp(al - m[ee_col])
        den = jax.ops.segment_sum(ex, ee_col, num_segments=n)
        return ex, den

    ex1, den1 = gat_scalars(0, 1)
    ex2, den2 = gat_scalars(2, 3)
    g = x2[ee_row]
    aggu1 = jax.ops.segment_sum(g * ex1[:, None], ee_col, num_segments=n)
    aggu2 = jax.ops.segment_sum(g * ex2[:, None], ee_col, num_segments=n)
    aggu1 = aggu1 / (den1 + 1e-16)[:, None]
    aggu2 = aggu2 / (den2 + 1e-16)[:, None]

    cat = jnp.concatenate([aggu1, aggu2], axis=1)
    wcat = jnp.concatenate([gat1_W, gat1re_W], axis=0)
    out = _mm_bias(cat, wcat, gat1_bias + gat1re_bias)
    return out[:num_ent] + ent_feats


# speedup vs baseline: 8.0786x; 6.7342x over previous
import functools

import jax
import jax.numpy as jnp
from jax import lax
from jax.experimental import pallas as pl
from jax.experimental.pallas import tpu as pltpu
from jax.experimental.pallas import tpu_sc as plsc

_SC_INFO = plsc.get_sparse_core_info()
_NC = _SC_INFO.num_cores
_NS = _SC_INFO.num_subcores
_NW = _NC * _NS
_BT = 512


def _mm_bias(x, w, b, block=2000):
    n, k = x.shape
    m = w.shape[1]
    assert n % block == 0

    def body(x_ref, w_ref, b_ref, o_ref):
        o_ref[...] = (
            jnp.dot(x_ref[...], w_ref[...], preferred_element_type=jnp.float32)
            + b_ref[...]
        )

    return pl.pallas_call(
        body,
        grid=(n // block,),
        in_specs=[
            pl.BlockSpec((block, k), lambda i: (i, 0)),
            pl.BlockSpec((k, m), lambda i: (0, 0)),
            pl.BlockSpec((1, m), lambda i: (0, 0)),
        ],
        out_specs=pl.BlockSpec((block, m), lambda i: (i, 0)),
        out_shape=jax.ShapeDtypeStruct((n, m), jnp.float32),
    )(x, w, b.reshape(1, m))


@functools.cache
def _sc_gather(d, epad):
    nb = epad // (_NW * _BT)
    mesh = plsc.VectorSubcoreMesh(core_axis_name="c", subcore_axis_name="s")

    @functools.partial(
        pl.kernel,
        mesh=mesh,
        out_type=jax.ShapeDtypeStruct((epad, d), jnp.float32),
        scratch_types=[
            pltpu.VMEM((_BT,), jnp.int32),
            pltpu.VMEM((_BT, d), jnp.float32),
            pltpu.SemaphoreType.DMA,
        ],
    )
    def k(table_hbm, idx_hbm, out_hbm, idx_v, rows_v, sem):
        wid = lax.axis_index("s") * _NC + lax.axis_index("c")

        def batch(i, carry):
            base = (wid * nb + i) * _BT
            pltpu.sync_copy(idx_hbm.at[pl.ds(base, _BT)], idx_v)
            pltpu.async_copy(table_hbm.at[idx_v], rows_v, sem).wait()
            pltpu.sync_copy(rows_v, out_hbm.at[pl.ds(base, _BT)])
            return carry

        lax.fori_loop(0, nb, batch, 0)

    return k


_BTS = 2048


@functools.cache
def _sc_scalar_gather(n_table, epad):
    nb = epad // (_NW * _BTS)
    mesh = plsc.VectorSubcoreMesh(core_axis_name="c", subcore_axis_name="s")

    @functools.partial(
        pl.kernel,
        mesh=mesh,
        out_type=jax.ShapeDtypeStruct((epad,), jnp.float32),
        scratch_types=[
            pltpu.VMEM((n_table,), jnp.float32),
            pltpu.VMEM((_BTS,), jnp.int32),
            pltpu.VMEM((_BTS,), jnp.float32),
        ],
        compiler_params=pltpu.CompilerParams(needs_layout_passes=False),
    )
    def k(table_hbm, idx_hbm, out_hbm, tab_v, idx_v, out_v):
        wid = lax.axis_index("s") * _NC + lax.axis_index("c")
        pltpu.sync_copy(table_hbm, tab_v)

        def batch(i, carry):
            base = (wid * nb + i) * _BTS
            pltpu.sync_copy(idx_hbm.at[pl.ds(base, _BTS)], idx_v)

            def vec(j, c2):
                ii = idx_v[pl.ds(j * 16, 16)]
                out_v[pl.ds(j * 16, 16)] = plsc.load_gather(tab_v, [ii])
                return c2

            lax.fori_loop(0, _BTS // 16, vec, 0)
            pltpu.sync_copy(out_v, out_hbm.at[pl.ds(base, _BTS)])
            return carry

        lax.fori_loop(0, nb, batch, 0)

    return k


def _gather_scalars(table, idx):
    e = idx.shape[0]
    step = _NW * _BTS
    epad = ((e + step - 1) // step) * step
    idx_p = jnp.pad(idx.astype(jnp.int32), (0, epad - e))
    return _sc_scalar_gather(table.shape[0], epad)(table, idx_p)[:e]


def _pad_len(e):
    step = _NW * _BT
    return ((e + step - 1) // step) * step


def _gather_rows(table, idx):
    e = idx.shape[0]
    epad = _pad_len(e)
    idx_p = jnp.pad(idx.astype(jnp.int32), (0, epad - e))
    return _sc_gather(table.shape[1], epad)(table, idx_p)


def kernel(attribute_triples, ent_edges, ent_edge_labels, val_edges,
           val_edge_labels, att_feats, val_feats, ent_feats, W, gcn1_W,
           gcn1_b, gcn2_W, gcn2_b, gat1_W, gat1_att_src, gat1_att_dst,
           gat1_bias, gat1re_W, gat1re_att_src, gat1re_att_dst, gat1re_bias):
    val = attribute_triples[:, 1]
    att = attribute_triples[:, 2]
    num_ent = ent_feats.shape[0]
    key_dim = ent_feats.shape[1]
    n = num_ent + val.shape[0]

    att_proj = att_feats @ W[:key_dim]
    val_proj = val_feats @ W[key_dim:]
    vfeat = _gather_rows(att_proj, att)[: val.shape[0]] \
        + _gather_rows(val_proj, val)[: val.shape[0]]
    x0 = jnp.concatenate([ent_feats, vfeat], axis=0)

    ve_row = val_edges[:, 0]
    ve_col = val_edges[:, 1]
    e_val = ve_row.shape[0]
    epad_v = _pad_len(e_val)
    ve_col_p = jnp.pad(ve_col, (0, epad_v - e_val), constant_values=n)

    deg = jax.ops.segment_sum(jnp.ones(e_val, jnp.float32), ve_col,
                              num_segments=n) + 1.0
    dinv = deg ** -0.5
    dinv2 = dinv * dinv

    def gcn_agg(x):
        y = x * dinv[:, None]
        g = _gather_rows(y, ve_row)
        s = jax.ops.segment_sum(g, ve_col_p, num_segments=n)
        return s * dinv[:, None] + x * dinv2[:, None]

    x1 = _mm_bias(gcn_agg(x0), gcn1_W, gcn1_b)
    x2 = _mm_bias(gcn_agg(x1), gcn2_W, gcn2_b)

    ee_row = ent_edges[:, 0]
    ee_col = ent_edges[:, 1]
    e_ent = ee_row.shape[0]
    epad_e = _pad_len(e_ent)
    ee_col_p = jnp.pad(ee_col, (0, epad_e - e_ent), constant_values=n)

    u1 = gat1_W @ gat1_att_src
    v1 = gat1_W @ gat1_att_dst
    u2 = gat1re_W @ gat1re_att_src
    v2 = gat1re_W @ gat1re_att_dst
    proj = x2 @ jnp.stack([u1, v1, u2, v2], axis=1)
    s1 = _gather_scalars(proj[:, 0], ee_row)
    d1 = _gather_scalars(proj[:, 1], ee_col)
    s2 = _gather_scalars(proj[:, 2], ee_row)
    d2 = _gather_scalars(proj[:, 3], ee_col)

    def leaky(a):
        return jnp.where(a > 0, a, 0.2 * a)

    ex1 = jnp.exp(leaky(s1 + d1))
    ex2 = jnp.exp(leaky(s2 + d2))
    exs1 = jnp.exp(leaky(proj[:, 0] + proj[:, 1]))
    exs2 = jnp.exp(leaky(proj[:, 2] + proj[:, 3]))

    g2 = _gather_rows(x2, ee_row)
    ex1_p = jnp.pad(ex1, (0, epad_e - e_ent))
    ex2_p = jnp.pad(ex2, (0, epad_e - e_ent))
    aggu1 = jax.ops.segment_sum(g2 * ex1_p[:, None], ee_col_p, num_segments=n)
    aggu2 = jax.ops.segment_sum(g2 * ex2_p[:, None], ee_col_p, num_segments=n)
    aggu1 = aggu1 + x2 * exs1[:, None]
    aggu2 = aggu2 + x2 * exs2[:, None]
    den1 = jax.ops.segment_sum(ex1_p, ee_col_p, num_segments=n) + exs1
    den2 = jax.ops.segment_sum(ex2_p, ee_col_p, num_segments=n) + exs2
    aggu1 = aggu1 / (den1 + 1e-16)[:, None]
    aggu2 = aggu2 / (den2 + 1e-16)[:, None]

    cat = jnp.concatenate([aggu1, aggu2], axis=1)
    wcat = jnp.concatenate([gat1_W, gat1re_W], axis=0)
    out = _mm_bias(cat, wcat, gat1_bias + gat1re_bias)
    return out[:num_ent] + ent_feats


# speedup vs baseline: 9.5424x; 1.1812x over previous
import functools

import jax
import jax.numpy as jnp
from jax import lax
from jax.experimental import pallas as pl
from jax.experimental.pallas import tpu as pltpu
from jax.experimental.pallas import tpu_sc as plsc

_SC_INFO = plsc.get_sparse_core_info()
_NC = _SC_INFO.num_cores
_NS = _SC_INFO.num_subcores
_NW = _NC * _NS
_BT = 512


def _mm_bias(x, w, b, block=2000):
    n, k = x.shape
    m = w.shape[1]
    assert n % block == 0

    def body(x_ref, w_ref, b_ref, o_ref):
        o_ref[...] = (
            jnp.dot(x_ref[...], w_ref[...], preferred_element_type=jnp.float32)
            + b_ref[...]
        )

    return pl.pallas_call(
        body,
        grid=(n // block,),
        in_specs=[
            pl.BlockSpec((block, k), lambda i: (i, 0)),
            pl.BlockSpec((k, m), lambda i: (0, 0)),
            pl.BlockSpec((1, m), lambda i: (0, 0)),
        ],
        out_specs=pl.BlockSpec((block, m), lambda i: (i, 0)),
        out_shape=jax.ShapeDtypeStruct((n, m), jnp.float32),
    )(x, w, b.reshape(1, m))


@functools.cache
def _sc_gather(d, epad):
    nb = epad // (_NW * _BT)
    mesh = plsc.VectorSubcoreMesh(core_axis_name="c", subcore_axis_name="s")

    @functools.partial(
        pl.kernel,
        mesh=mesh,
        out_type=jax.ShapeDtypeStruct((epad, d), jnp.float32),
        scratch_types=[
            pltpu.VMEM((_BT,), jnp.int32),
            pltpu.VMEM((_BT, d), jnp.float32),
            pltpu.SemaphoreType.DMA,
        ],
    )
    def k(table_hbm, idx_hbm, out_hbm, idx_v, rows_v, sem):
        wid = lax.axis_index("s") * _NC + lax.axis_index("c")

        def batch(i, carry):
            base = (wid * nb + i) * _BT
            pltpu.sync_copy(idx_hbm.at[pl.ds(base, _BT)], idx_v)
            pltpu.async_copy(table_hbm.at[idx_v], rows_v, sem).wait()
            pltpu.sync_copy(rows_v, out_hbm.at[pl.ds(base, _BT)])
            return carry

        lax.fori_loop(0, nb, batch, 0)

    return k


_BTS = 2048


@functools.cache
def _sc_scalar_gather(n_table, epad, dtype=jnp.float32):
    nb = epad // (_NW * _BTS)
    mesh = plsc.VectorSubcoreMesh(core_axis_name="c", subcore_axis_name="s")

    @functools.partial(
        pl.kernel,
        mesh=mesh,
        out_type=jax.ShapeDtypeStruct((epad,), dtype),
        scratch_types=[
            pltpu.VMEM((n_table,), dtype),
            pltpu.VMEM((_BTS,), jnp.int32),
            pltpu.VMEM((_BTS,), dtype),
        ],
        compiler_params=pltpu.CompilerParams(needs_layout_passes=False),
    )
    def k(table_hbm, idx_hbm, out_hbm, tab_v, idx_v, out_v):
        wid = lax.axis_index("s") * _NC + lax.axis_index("c")
        pltpu.sync_copy(table_hbm, tab_v)

        def batch(i, carry):
            base = (wid * nb + i) * _BTS
            pltpu.sync_copy(idx_hbm.at[pl.ds(base, _BTS)], idx_v)

            def vec(j, c2):
                ii = idx_v[pl.ds(j * 16, 16)]
                out_v[pl.ds(j * 16, 16)] = plsc.load_gather(tab_v, [ii])
                return c2

            lax.fori_loop(0, _BTS // 16, vec, 0)
            pltpu.sync_copy(out_v, out_hbm.at[pl.ds(base, _BTS)])
            return carry

        lax.fori_loop(0, nb, batch, 0)

    return k


def _gather_scalars(table, idx):
    e = idx.shape[0]
    step = _NW * _BTS
    epad = ((e + step - 1) // step) * step
    idx_p = jnp.pad(idx.astype(jnp.int32), (0, epad - e))
    return _sc_scalar_gather(table.shape[0], epad, table.dtype.type)(
        table, idx_p)[:e]


def _gather_scalars_i32(table, idx):
    return _gather_scalars(table.astype(jnp.int32), idx)


def _pad_len(e):
    step = _NW * _BT
    return ((e + step - 1) // step) * step


def _spread_pad(idx, epad, n_rows):
    e = idx.shape[0]
    pad = jnp.arange(epad - e, dtype=jnp.int32) % jnp.int32(n_rows)
    return jnp.concatenate([idx.astype(jnp.int32), pad])


def _gather_rows(table, idx):
    e = idx.shape[0]
    epad = _pad_len(e)
    idx_p = _spread_pad(idx, epad, table.shape[0])
    return _sc_gather(table.shape[1], epad)(table, idx_p)


def kernel(attribute_triples, ent_edges, ent_edge_labels, val_edges,
           val_edge_labels, att_feats, val_feats, ent_feats, W, gcn1_W,
           gcn1_b, gcn2_W, gcn2_b, gat1_W, gat1_att_src, gat1_att_dst,
           gat1_bias, gat1re_W, gat1re_att_src, gat1re_att_dst, gat1re_bias):
    val = attribute_triples[:, 1]
    att = attribute_triples[:, 2]
    num_ent = ent_feats.shape[0]
    key_dim = ent_feats.shape[1]
    n = num_ent + val.shape[0]

    att_proj = att_feats @ W[:key_dim]
    val_proj = val_feats @ W[key_dim:]
    vfeat = _gather_rows(att_proj, att)[: val.shape[0]] \
        + _gather_rows(val_proj, val)[: val.shape[0]]
    x0 = jnp.concatenate([ent_feats, vfeat], axis=0)

    ve_row = val_edges[:, 0]
    ve_col = val_edges[:, 1]
    e_val = ve_row.shape[0]
    epad_v = _pad_len(e_val)
    ve_col_p = jnp.pad(ve_col, (0, epad_v - e_val), constant_values=n)

    deg = jax.ops.segment_sum(jnp.ones(e_val, jnp.float32), ve_col,
                              num_segments=n) + 1.0
    dinv = deg ** -0.5
    dinv2 = dinv * dinv

    def gcn_agg(x):
        y = x * dinv[:, None]
        g = _gather_rows(y, ve_row)
        s = jax.ops.segment_sum(g, ve_col_p, num_segments=n)
        return s * dinv[:, None] + x * dinv2[:, None]

    x1 = _mm_bias(gcn_agg(x0), gcn1_W, gcn1_b)
    x2 = _mm_bias(gcn_agg(x1), gcn2_W, gcn2_b)

    ee_row = ent_edges[:, 0]
    ee_col = ent_edges[:, 1]
    e_ent = ee_row.shape[0]
    epad_e = _pad_len(e_ent)
    ee_col_p = jnp.pad(ee_col, (0, epad_e - e_ent), constant_values=n)

    u1 = gat1_W @ gat1_att_src
    v1 = gat1_W @ gat1_att_dst
    u2 = gat1re_W @ gat1re_att_src
    v2 = gat1re_W @ gat1re_att_dst
    proj = x2 @ jnp.stack([u1, v1, u2, v2], axis=1)
    s1 = _gather_scalars(proj[:, 0], ee_row)
    d1 = _gather_scalars(proj[:, 1], ee_col)
    s2 = _gather_scalars(proj[:, 2], ee_row)
    d2 = _gather_scalars(proj[:, 3], ee_col)

    def leaky(a):
        return jnp.where(a > 0, a, 0.2 * a)

    ex1 = jnp.exp(leaky(s1 + d1))
    ex2 = jnp.exp(leaky(s2 + d2))
    exs1 = jnp.exp(leaky(proj[:, 0] + proj[:, 1]))
    exs2 = jnp.exp(leaky(proj[:, 2] + proj[:, 3]))

    g2 = _gather_rows(x2, ee_row)
    ex1_p = jnp.pad(ex1, (0, epad_e - e_ent))
    ex2_p = jnp.pad(ex2, (0, epad_e - e_ent))
    mcat = jnp.concatenate([g2 * ex1_p[:, None], g2 * ex2_p[:, None]], axis=1)
    aggu = jax.ops.segment_sum(mcat, ee_col_p, num_segments=n)
    aggu1, aggu2 = aggu[:, :key_dim], aggu[:, key_dim:]
    aggu1 = aggu1 + x2 * exs1[:, None]
    aggu2 = aggu2 + x2 * exs2[:, None]
    den1 = jax.ops.segment_sum(ex1_p, ee_col_p, num_segments=n) + exs1
    den2 = jax.ops.segment_sum(ex2_p, ee_col_p, num_segments=n) + exs2
    aggu1 = aggu1 / (den1 + 1e-16)[:, None]
    aggu2 = aggu2 / (den2 + 1e-16)[:, None]

    cat = jnp.concatenate([aggu1, aggu2], axis=1)
    wcat = jnp.concatenate([gat1_W, gat1re_W], axis=0)
    out = _mm_bias(cat, wcat, gat1_bias + gat1re_bias)
    return out[:num_ent] + ent_feats


# speedup vs baseline: 11.6977x; 1.2259x over previous
import functools

import jax
import jax.numpy as jnp
from jax import lax
from jax.experimental import pallas as pl
from jax.experimental.pallas import tpu as pltpu
from jax.experimental.pallas import tpu_sc as plsc

_SC_INFO = plsc.get_sparse_core_info()
_NC = _SC_INFO.num_cores
_NS = _SC_INFO.num_subcores
_NW = _NC * _NS
_BT = 512


def _mm_bias(x, w, b, block=2000):
    n, k = x.shape
    m = w.shape[1]
    assert n % block == 0

    def body(x_ref, w_ref, b_ref, o_ref):
        o_ref[...] = (
            jnp.dot(x_ref[...], w_ref[...], preferred_element_type=jnp.float32)
            + b_ref[...]
        )

    return pl.pallas_call(
        body,
        grid=(n // block,),
        in_specs=[
            pl.BlockSpec((block, k), lambda i: (i, 0)),
            pl.BlockSpec((k, m), lambda i: (0, 0)),
            pl.BlockSpec((1, m), lambda i: (0, 0)),
        ],
        out_specs=pl.BlockSpec((block, m), lambda i: (i, 0)),
        out_shape=jax.ShapeDtypeStruct((n, m), jnp.float32),
    )(x, w, b.reshape(1, m))


@functools.cache
def _sc_gather(d, epad):
    nb = epad // (_NW * _BT)
    mesh = plsc.VectorSubcoreMesh(core_axis_name="c", subcore_axis_name="s")

    @functools.partial(
        pl.kernel,
        mesh=mesh,
        out_type=jax.ShapeDtypeStruct((epad, d), jnp.float32),
        scratch_types=[
            pltpu.VMEM((_BT,), jnp.int32),
            pltpu.VMEM((_BT, d), jnp.float32),
            pltpu.SemaphoreType.DMA,
        ],
    )
    def k(table_hbm, idx_hbm, out_hbm, idx_v, rows_v, sem):
        wid = lax.axis_index("s") * _NC + lax.axis_index("c")

        def batch(i, carry):
            base = (wid * nb + i) * _BT
            pltpu.sync_copy(idx_hbm.at[pl.ds(base, _BT)], idx_v)
            pltpu.async_copy(table_hbm.at[idx_v], rows_v, sem).wait()
            pltpu.sync_copy(rows_v, out_hbm.at[pl.ds(base, _BT)])
            return carry

        lax.fori_loop(0, nb, batch, 0)

    return k


_BTS = 2048


@functools.cache
def _sc_scalar_gather(n_table, epad, dtype=jnp.float32):
    nb = epad // (_NW * _BTS)
    mesh = plsc.VectorSubcoreMesh(core_axis_name="c", subcore_axis_name="s")

    @functools.partial(
        pl.kernel,
        mesh=mesh,
        out_type=jax.ShapeDtypeStruct((epad,), dtype),
        scratch_types=[
            pltpu.VMEM((n_table,), dtype),
            pltpu.VMEM((_BTS,), jnp.int32),
            pltpu.VMEM((_BTS,), dtype),
        ],
        compiler_params=pltpu.CompilerParams(needs_layout_passes=False),
    )
    def k(table_hbm, idx_hbm, out_hbm, tab_v, idx_v, out_v):
        wid = lax.axis_index("s") * _NC + lax.axis_index("c")
        pltpu.sync_copy(table_hbm, tab_v)

        def batch(i, carry):
            base = (wid * nb + i) * _BTS
            pltpu.sync_copy(idx_hbm.at[pl.ds(base, _BTS)], idx_v)

            def vec(j, c2):
                ii = idx_v[pl.ds(j * 16, 16)]
                out_v[pl.ds(j * 16, 16)] = plsc.load_gather(tab_v, [ii])
                return c2

            lax.fori_loop(0, _BTS // 16, vec, 0)
            pltpu.sync_copy(out_v, out_hbm.at[pl.ds(base, _BTS)])
            return carry

        lax.fori_loop(0, nb, batch, 0)

    return k


def _gather_scalars(table, idx):
    e = idx.shape[0]
    step = _NW * _BTS
    epad = ((e + step - 1) // step) * step
    idx_p = jnp.pad(idx.astype(jnp.int32), (0, epad - e))
    return _sc_scalar_gather(table.shape[0], epad, table.dtype.type)(
        table, idx_p)[:e]


def _gather_scalars_i32(table, idx):
    return _gather_scalars(table.astype(jnp.int32), idx)


def _pad_len(e):
    step = _NW * _BT
    return ((e + step - 1) // step) * step


_CHK = 13568
_DUM = 128
_NCH = 8
_STRIPE = _CHK // 16


@functools.cache
def _sc_gcn_agg(n_pad, epad):
    nblk = epad // _BT
    npass = (_NCH + 1) // 2
    n_out = _NCH * _CHK
    assert nblk % 16 == 0
    mesh = plsc.VectorSubcoreMesh(core_axis_name="c", subcore_axis_name="s")

    @functools.partial(
        pl.kernel,
        mesh=mesh,
        out_type=jax.ShapeDtypeStruct((n_out, 128), jnp.float32),
        scratch_types=[
            pltpu.VMEM_SHARED((_CHK + _DUM, 128), jnp.float32),
            pltpu.VMEM((_BT,), jnp.int32),
            pltpu.VMEM((_BT,), jnp.int32),
            pltpu.VMEM((1056,), jnp.int32),
            pltpu.VMEM((1056,), jnp.int32),
            pltpu.VMEM((4, 128), jnp.int32),
            pltpu.VMEM((128, 128), jnp.float32),
            pltpu.SemaphoreType.DMA,
        ],
        compiler_params=pltpu.CompilerParams(needs_layout_passes=False),
    )
    def k(y_hbm, rows_hbm, cols_hbm, out_hbm, acc, rblk, cblk, fr, fs,
          s2d, rows_v, sem):
        c = lax.axis_index("c")
        s = lax.axis_index("s")
        t = s
        lane0 = lax.iota(jnp.int32, 16)
        for m in range(66):
            fr[pl.ds(m * 16, 16)] = lane0 + m * 16

        def seg_fire(j, rbase):
            for m in range(8):
                s2d[j, pl.ds(m * 16, 16)] = fs[pl.ds(rbase + m * 16, 16)]
            pltpu.async_copy(
                y_hbm.at[fr.at[pl.ds(rbase, 128)]], rows_v, sem).wait()
            pltpu.sync_copy(rows_v, acc.at[s2d.at[j]], add=True)

        def one_pass(j, carry):
            ch = c + 2 * j
            base = ch * _CHK
            live = ch < _NCH

            @pl.when(live)
            def _():
                pltpu.sync_copy(
                    y_hbm.at[pl.ds(base + t * _STRIPE, _STRIPE)],
                    acc.at[pl.ds(t * _STRIPE, _STRIPE)])
            plsc.subcore_barrier()

            def block(b, f):
                blk = (b * 16 + t) * _BT
                pltpu.sync_copy(rows_hbm.at[pl.ds(blk, _BT)], rblk)
                pltpu.sync_copy(cols_hbm.at[pl.ds(blk, _BT)], cblk)

                def compress_one(kk, f):
                    c16 = cblk[pl.ds(kk * 16, 16)]
                    li = c16 - base
                    msk = (li >= 0) & (li < _CHK)
                    r16 = rblk[pl.ds(kk * 16, 16)]
                    plsc.store_compressed(fr.at[pl.ds(f, 16)], r16, mask=msk)
                    plsc.store_compressed(fs.at[pl.ds(f, 16)], li, mask=msk)
                    return f + jnp.max(plsc.all_reduce_population_count(msk))

                f = lax.fori_loop(0, _BT // 16, compress_one, f)
                fired = f >= 512

                @pl.when(fired)
                def _():
                    for j4 in range(4):
                        seg_fire(j4, j4 * 128)
                    for m in range(32):
                        fr[pl.ds(m * 16, 16)] = fr[pl.ds(512 + m * 16, 16)]
                        fs[pl.ds(m * 16, 16)] = fs[pl.ds(512 + m * 16, 16)]

                return jnp.where(fired, f - 512, f)

            f = lax.fori_loop(0, nblk // 16, block, jnp.int32(0))

            lane = lax.iota(jnp.int32, 16)
            for m in range(8):
                fs[pl.ds(f + m * 16, 16)] = _CHK + ((lane + m * 16) & (_DUM - 1))

            def flush_seg(j2, carry2):
                seg_fire(j2, j2 * 128)
                return carry2

            lax.fori_loop(0, (f + 127) >> 7, flush_seg, 0)
            plsc.subcore_barrier()

            @pl.when(live)
            def _():
                pltpu.sync_copy(
                    acc.at[pl.ds(t * _STRIPE, _STRIPE)],
                    out_hbm.at[pl.ds(base + t * _STRIPE, _STRIPE)])
            plsc.subcore_barrier()
            return carry

        lax.fori_loop(0, npass, one_pass, 0)

    return k


def _gcn_agg(y_init, rows, cols, n):
    e = rows.shape[0]
    epad = _pad_len(e)
    n_out = _NCH * _CHK
    yp = jnp.pad(y_init, ((0, n_out - n), (0, 0)))
    rows_p = _spread_pad(rows, epad, n)
    cols_p = jnp.pad(cols.astype(jnp.int32), (0, epad - e),
                     constant_values=n_out + _CHK)
    out = _sc_gcn_agg(n_out, epad)(yp, rows_p, cols_p)
    return out[:n]


def _spread_pad(idx, epad, n_rows):
    e = idx.shape[0]
    pad = jnp.arange(epad - e, dtype=jnp.int32) % jnp.int32(n_rows)
    return jnp.concatenate([idx.astype(jnp.int32), pad])


def _gather_rows(table, idx):
    e = idx.shape[0]
    epad = _pad_len(e)
    idx_p = _spread_pad(idx, epad, table.shape[0])
    return _sc_gather(table.shape[1], epad)(table, idx_p)


def kernel(attribute_triples, ent_edges, ent_edge_labels, val_edges,
           val_edge_labels, att_feats, val_feats, ent_feats, W, gcn1_W,
           gcn1_b, gcn2_W, gcn2_b, gat1_W, gat1_att_src, gat1_att_dst,
           gat1_bias, gat1re_W, gat1re_att_src, gat1re_att_dst, gat1re_bias):
    val = attribute_triples[:, 1]
    att = attribute_triples[:, 2]
    num_ent = ent_feats.shape[0]
    key_dim = ent_feats.shape[1]
    n = num_ent + val.shape[0]

    att_proj = att_feats @ W[:key_dim]
    val_proj = val_feats @ W[key_dim:]
    vfeat = _gather_rows(att_proj, att)[: val.shape[0]] \
        + _gather_rows(val_proj, val)[: val.shape[0]]
    x0 = jnp.concatenate([ent_feats, vfeat], axis=0)

    ve_row = val_edges[:, 0]
    ve_col = val_edges[:, 1]
    e_val = ve_row.shape[0]
    epad_v = _pad_len(e_val)
    ve_col_p = jnp.pad(ve_col, (0, epad_v - e_val), constant_values=n)

    deg = jax.ops.segment_sum(jnp.ones(e_val, jnp.float32), ve_col,
                              num_segments=n) + 1.0
    dinv = deg ** -0.5
    dinv2 = dinv * dinv

    def gcn_agg(x):
        y = x * dinv[:, None]
        return _gcn_agg(y, ve_row, ve_col, n) * dinv[:, None]

    x1 = _mm_bias(gcn_agg(x0), gcn1_W, gcn1_b)
    x2 = _mm_bias(gcn_agg(x1), gcn2_W, gcn2_b)

    ee_row = ent_edges[:, 0]
    ee_col = ent_edges[:, 1]
    e_ent = ee_row.shape[0]
    epad_e = _pad_len(e_ent)
    ee_col_p = jnp.pad(ee_col, (0, epad_e - e_ent), constant_values=n)

    u1 = gat1_W @ gat1_att_src
    v1 = gat1_W @ gat1_att_dst
    u2 = gat1re_W @ gat1re_att_src
    v2 = gat1re_W @ gat1re_att_dst
    proj = x2 @ jnp.stack([u1, v1, u2, v2], axis=1)
    s1 = _gather_scalars(proj[:, 0], ee_row)
    d1 = _gather_scalars(proj[:, 1], ee_col)
    s2 = _gather_scalars(proj[:, 2], ee_row)
    d2 = _gather_scalars(proj[:, 3], ee_col)

    def leaky(a):
        return jnp.where(a > 0, a, 0.2 * a)

    ex1 = jnp.exp(leaky(s1 + d1))
    ex2 = jnp.exp(leaky(s2 + d2))
    exs1 = jnp.exp(leaky(proj[:, 0] + proj[:, 1]))
    exs2 = jnp.exp(leaky(proj[:, 2] + proj[:, 3]))

    g2 = _gather_rows(x2, ee_row)
    ex1_p = jnp.pad(ex1, (0, epad_e - e_ent))
    ex2_p = jnp.pad(ex2, (0, epad_e - e_ent))
    mcat = jnp.concatenate([g2 * ex1_p[:, None], g2 * ex2_p[:, None]], axis=1)
    aggu = jax.ops.segment_sum(mcat, ee_col_p, num_segments=n)
    aggu1, aggu2 = aggu[:, :key_dim], aggu[:, key_dim:]
    aggu1 = aggu1 + x2 * exs1[:, None]
    aggu2 = aggu2 + x2 * exs2[:, None]
    den1 = jax.ops.segment_sum(ex1_p, ee_col_p, num_segments=n) + exs1
    den2 = jax.ops.segment_sum(ex2_p, ee_col_p, num_segments=n) + exs2
    aggu1 = aggu1 / (den1 + 1e-16)[:, None]
    aggu2 = aggu2 / (den2 + 1e-16)[:, None]

    cat = jnp.concatenate([aggu1, aggu2], axis=1)
    wcat = jnp.concatenate([gat1_W, gat1re_W], axis=0)
    out = _mm_bias(cat, wcat, gat1_bias + gat1re_bias)
    return out[:num_ent] + ent_feats


# speedup vs baseline: 12.1997x; 1.0429x over previous
import functools

import jax
import jax.numpy as jnp
from jax import lax
from jax.experimental import pallas as pl
from jax.experimental.pallas import tpu as pltpu
from jax.experimental.pallas import tpu_sc as plsc

_SC_INFO = plsc.get_sparse_core_info()
_NC = _SC_INFO.num_cores
_NS = _SC_INFO.num_subcores
_NW = _NC * _NS
_BT = 512


def _mm_bias(x, w, b, block=2000):
    n, k = x.shape
    m = w.shape[1]
    assert n % block == 0

    def body(x_ref, w_ref, b_ref, o_ref):
        o_ref[...] = (
            jnp.dot(x_ref[...], w_ref[...], preferred_element_type=jnp.float32)
            + b_ref[...]
        )

    return pl.pallas_call(
        body,
        grid=(n // block,),
        in_specs=[
            pl.BlockSpec((block, k), lambda i: (i, 0)),
            pl.BlockSpec((k, m), lambda i: (0, 0)),
            pl.BlockSpec((1, m), lambda i: (0, 0)),
        ],
        out_specs=pl.BlockSpec((block, m), lambda i: (i, 0)),
        out_shape=jax.ShapeDtypeStruct((n, m), jnp.float32),
    )(x, w, b.reshape(1, m))


@functools.cache
def _sc_gather(d, epad):
    nb = epad // (_NW * _BT)
    mesh = plsc.VectorSubcoreMesh(core_axis_name="c", subcore_axis_name="s")

    @functools.partial(
        pl.kernel,
        mesh=mesh,
        out_type=jax.ShapeDtypeStruct((epad, d), jnp.float32),
        scratch_types=[
            pltpu.VMEM((_BT,), jnp.int32),
            pltpu.VMEM((_BT, d), jnp.float32),
            pltpu.SemaphoreType.DMA,
        ],
    )
    def k(table_hbm, idx_hbm, out_hbm, idx_v, rows_v, sem):
        wid = lax.axis_index("s") * _NC + lax.axis_index("c")

        def batch(i, carry):
            base = (wid * nb + i) * _BT
            pltpu.sync_copy(idx_hbm.at[pl.ds(base, _BT)], idx_v)
            pltpu.async_copy(table_hbm.at[idx_v], rows_v, sem).wait()
            pltpu.sync_copy(rows_v, out_hbm.at[pl.ds(base, _BT)])
            return carry

        lax.fori_loop(0, nb, batch, 0)

    return k


_BTS = 2048


@functools.cache
def _sc_scalar_gather(n_table, epad, dtype=jnp.float32):
    nb = epad // (_NW * _BTS)
    mesh = plsc.VectorSubcoreMesh(core_axis_name="c", subcore_axis_name="s")

    @functools.partial(
        pl.kernel,
        mesh=mesh,
        out_type=jax.ShapeDtypeStruct((epad,), dtype),
        scratch_types=[
            pltpu.VMEM((n_table,), dtype),
            pltpu.VMEM((_BTS,), jnp.int32),
            pltpu.VMEM((_BTS,), dtype),
        ],
        compiler_params=pltpu.CompilerParams(needs_layout_passes=False),
    )
    def k(table_hbm, idx_hbm, out_hbm, tab_v, idx_v, out_v):
        wid = lax.axis_index("s") * _NC + lax.axis_index("c")
        pltpu.sync_copy(table_hbm, tab_v)

        def batch(i, carry):
            base = (wid * nb + i) * _BTS
            pltpu.sync_copy(idx_hbm.at[pl.ds(base, _BTS)], idx_v)

            def vec(j, c2):
                ii = idx_v[pl.ds(j * 16, 16)]
                out_v[pl.ds(j * 16, 16)] = plsc.load_gather(tab_v, [ii])
                return c2

            lax.fori_loop(0, _BTS // 16, vec, 0)
            pltpu.sync_copy(out_v, out_hbm.at[pl.ds(base, _BTS)])
            return carry

        lax.fori_loop(0, nb, batch, 0)

    return k


def _gather_scalars(table, idx):
    e = idx.shape[0]
    step = _NW * _BTS
    epad = ((e + step - 1) // step) * step
    idx_p = jnp.pad(idx.astype(jnp.int32), (0, epad - e))
    return _sc_scalar_gather(table.shape[0], epad, table.dtype.type)(
        table, idx_p)[:e]


def _gather_scalars_i32(table, idx):
    return _gather_scalars(table.astype(jnp.int32), idx)


def _pad_len(e):
    step = _NW * _BT
    return ((e + step - 1) // step) * step


_CHK = 13568
_DUM = 128
_NCH = 8
_STRIPE = _CHK // 16


@functools.cache
def _sc_gcn_agg(n_pad, epad):
    nblk = epad // _BT
    npass = (_NCH + 1) // 2
    n_out = _NCH * _CHK
    assert nblk % 16 == 0
    mesh = plsc.VectorSubcoreMesh(core_axis_name="c", subcore_axis_name="s")

    @functools.partial(
        pl.kernel,
        mesh=mesh,
        out_type=jax.ShapeDtypeStruct((n_out, 128), jnp.float32),
        scratch_types=[
            pltpu.VMEM_SHARED((_CHK + _DUM, 128), jnp.float32),
            pltpu.VMEM((_BT,), jnp.int32),
            pltpu.VMEM((_BT,), jnp.int32),
            pltpu.VMEM((1056,), jnp.int32),
            pltpu.VMEM((1056,), jnp.int32),
            pltpu.VMEM((4, 128), jnp.int32),
            pltpu.VMEM((128, 128), jnp.float32),
            pltpu.SemaphoreType.DMA,
        ],
        compiler_params=pltpu.CompilerParams(needs_layout_passes=False),
    )
    def k(y_hbm, rows_hbm, cols_hbm, out_hbm, acc, rblk, cblk, fr, fs,
          s2d, rows_v, sem):
        c = lax.axis_index("c")
        s = lax.axis_index("s")
        t = s
        lane0 = lax.iota(jnp.int32, 16)
        for m in range(66):
            fr[pl.ds(m * 16, 16)] = lane0 + m * 16

        def seg_fire(j, rbase):
            for m in range(8):
                s2d[j, pl.ds(m * 16, 16)] = fs[pl.ds(rbase + m * 16, 16)]
            pltpu.async_copy(
                y_hbm.at[fr.at[pl.ds(rbase, 128)]], rows_v, sem).wait()
            pltpu.sync_copy(rows_v, acc.at[s2d.at[j]], add=True)

        def one_pass(j, carry):
            ch = c + 2 * j
            base = ch * _CHK
            live = ch < _NCH

            @pl.when(live)
            def _():
                pltpu.sync_copy(
                    y_hbm.at[pl.ds(base + t * _STRIPE, _STRIPE)],
                    acc.at[pl.ds(t * _STRIPE, _STRIPE)])
            plsc.subcore_barrier()

            def block(b, f):
                blk = (b * 16 + t) * _BT
                pltpu.sync_copy(rows_hbm.at[pl.ds(blk, _BT)], rblk)
                pltpu.sync_copy(cols_hbm.at[pl.ds(blk, _BT)], cblk)

                def compress_one(kk, f):
                    c16 = cblk[pl.ds(kk * 16, 16)]
                    li = c16 - base
                    msk = (li >= 0) & (li < _CHK)
                    r16 = rblk[pl.ds(kk * 16, 16)]
                    plsc.store_compressed(fr.at[pl.ds(f, 16)], r16, mask=msk)
                    plsc.store_compressed(fs.at[pl.ds(f, 16)], li, mask=msk)
                    return f + jnp.max(plsc.all_reduce_population_count(msk))

                f = lax.fori_loop(0, _BT // 16, compress_one, f)
                fired = f >= 512

                @pl.when(fired)
                def _():
                    for j4 in range(4):
                        seg_fire(j4, j4 * 128)
                    for m in range(32):
                        fr[pl.ds(m * 16, 16)] = fr[pl.ds(512 + m * 16, 16)]
                        fs[pl.ds(m * 16, 16)] = fs[pl.ds(512 + m * 16, 16)]

                return jnp.where(fired, f - 512, f)

            f = lax.fori_loop(0, nblk // 16, block, jnp.int32(0))

            lane = lax.iota(jnp.int32, 16)
            for m in range(8):
                fs[pl.ds(f + m * 16, 16)] = _CHK + ((lane + m * 16) & (_DUM - 1))

            def flush_seg(j2, carry2):
                seg_fire(j2, j2 * 128)
                return carry2

            lax.fori_loop(0, (f + 127) >> 7, flush_seg, 0)
            plsc.subcore_barrier()

            @pl.when(live)
            def _():
                pltpu.sync_copy(
                    acc.at[pl.ds(t * _STRIPE, _STRIPE)],
                    out_hbm.at[pl.ds(base + t * _STRIPE, _STRIPE)])
            plsc.subcore_barrier()
            return carry

        lax.fori_loop(0, npass, one_pass, 0)

    return k


_CHK2 = 11264
_NCH2 = 9
_STRIPE2 = _CHK2 // 16


@functools.cache
def _sc_gat_agg(n_pad, epad):
    nblk = epad // _BT
    npass = (_NCH2 + 1) // 2
    n_out = _NCH2 * _CHK2
    assert nblk % 16 == 0
    mesh = plsc.VectorSubcoreMesh(core_axis_name="c", subcore_axis_name="s")

    @functools.partial(
        pl.kernel,
        mesh=mesh,
        out_type=jax.ShapeDtypeStruct((n_out, 128), jnp.float32),
        scratch_types=[
            pltpu.VMEM_SHARED((_CHK2 + _DUM, 128), jnp.float32),
            pltpu.VMEM((_BT,), jnp.int32),
            pltpu.VMEM((_BT,), jnp.int32),
            pltpu.VMEM((_BT,), jnp.float32),
            pltpu.VMEM((1056,), jnp.int32),
            pltpu.VMEM((1056,), jnp.int32),
            pltpu.VMEM((1056,), jnp.float32),
            pltpu.VMEM((4, 128), jnp.int32),
            pltpu.VMEM((128, 128), jnp.float32),
            pltpu.VMEM((128, 128), jnp.float32),
            pltpu.SemaphoreType.DMA,
        ],
        compiler_params=pltpu.CompilerParams(needs_layout_passes=False),
    )
    def k(y_hbm, init_hbm, rows_hbm, cols_hbm, w_hbm, out_hbm, acc, rblk,
          cblk, wblk, fr, fs, fw, s2d, rows_v, sc_v, sem):
        c = lax.axis_index("c")
        t = lax.axis_index("s")
        lane0 = lax.iota(jnp.int32, 16)
        for m in range(66):
            fr[pl.ds(m * 16, 16)] = lane0 + m * 16
            fw[pl.ds(m * 16, 16)] = jnp.zeros((16,), jnp.float32)

        def seg_fire(j, rbase):
            for m in range(8):
                s2d[j, pl.ds(m * 16, 16)] = fs[pl.ds(rbase + m * 16, 16)]
            pltpu.async_copy(
                y_hbm.at[fr.at[pl.ds(rbase, 128)]], rows_v, sem).wait()

            def scale_row(r, c2):
                w = plsc.load_gather(
                    fw, [jnp.broadcast_to(rbase + r, (16,))])
                for m in range(8):
                    sc_v[r, pl.ds(m * 16, 16)] = (
                        rows_v[r, pl.ds(m * 16, 16)] * w)
                return c2

            lax.fori_loop(0, 128, scale_row, 0)
            pltpu.sync_copy(sc_v, acc.at[s2d.at[j]], add=True)

        def one_pass(j, carry):
            ch = c + 2 * j
            base = ch * _CHK2
            live = ch < _NCH2

            @pl.when(live)
            def _():
                pltpu.sync_copy(
                    init_hbm.at[pl.ds(base + t * _STRIPE2, _STRIPE2)],
                    acc.at[pl.ds(t * _STRIPE2, _STRIPE2)])
            plsc.subcore_barrier()

            def block(b, f):
                blk = (b * 16 + t) * _BT
                pltpu.sync_copy(rows_hbm.at[pl.ds(blk, _BT)], rblk)
                pltpu.sync_copy(cols_hbm.at[pl.ds(blk, _BT)], cblk)
                pltpu.sync_copy(w_hbm.at[pl.ds(blk, _BT)], wblk)

                def compress_one(kk, f):
                    c16 = cblk[pl.ds(kk * 16, 16)]
                    li = c16 - base
                    msk = (li >= 0) & (li < _CHK2)
                    r16 = rblk[pl.ds(kk * 16, 16)]
                    w16 = wblk[pl.ds(kk * 16, 16)]
                    plsc.store_compressed(fr.at[pl.ds(f, 16)], r16, mask=msk)
                    plsc.store_compressed(fs.at[pl.ds(f, 16)], li, mask=msk)
                    plsc.store_compressed(fw.at[pl.ds(f, 16)], w16, mask=msk)
                    return f + jnp.max(plsc.all_reduce_population_count(msk))

                f = lax.fori_loop(0, _BT // 16, compress_one, f)
                fired = f >= 512

                @pl.when(fired)
                def _():
                    for j4 in range(4):
                        seg_fire(j4, j4 * 128)
                    for m in range(32):
                        fr[pl.ds(m * 16, 16)] = fr[pl.ds(512 + m * 16, 16)]
                        fs[pl.ds(m * 16, 16)] = fs[pl.ds(512 + m * 16, 16)]
                        fw[pl.ds(m * 16, 16)] = fw[pl.ds(512 + m * 16, 16)]

                return jnp.where(fired, f - 512, f)

            f = lax.fori_loop(0, nblk // 16, block, jnp.int32(0))

            lane = lax.iota(jnp.int32, 16)
            for m in range(8):
                fs[pl.ds(f + m * 16, 16)] = (
                    _CHK2 + ((lane + m * 16) & (_DUM - 1)))

            def flush_seg(j2, carry2):
                seg_fire(j2, j2 * 128)
                return carry2

            lax.fori_loop(0, (f + 127) >> 7, flush_seg, 0)
            plsc.subcore_barrier()

            @pl.when(live)
            def _():
                pltpu.sync_copy(
                    acc.at[pl.ds(t * _STRIPE2, _STRIPE2)],
                    out_hbm.at[pl.ds(base + t * _STRIPE2, _STRIPE2)])
            plsc.subcore_barrier()
            return carry

        lax.fori_loop(0, npass, one_pass, 0)

    return k


def _gat_agg(y, init, rows, cols, w, n):
    e = rows.shape[0]
    epad = _pad_len(e)
    n_out = _NCH2 * _CHK2
    init_p = jnp.pad(init, ((0, n_out - n), (0, 0)))
    rows_p = _spread_pad(rows, epad, n)
    cols_p = jnp.pad(cols.astype(jnp.int32), (0, epad - e),
                     constant_values=n_out + _CHK2)
    w_p = jnp.pad(w, (0, epad - e))
    out = _sc_gat_agg(n_out, epad)(y, init_p, rows_p, cols_p, w_p)
    return out[:n]


def _gcn_agg(y_init, rows, cols, n):
    e = rows.shape[0]
    epad = _pad_len(e)
    n_out = _NCH * _CHK
    yp = jnp.pad(y_init, ((0, n_out - n), (0, 0)))
    rows_p = _spread_pad(rows, epad, n)
    cols_p = jnp.pad(cols.astype(jnp.int32), (0, epad - e),
                     constant_values=n_out + _CHK)
    out = _sc_gcn_agg(n_out, epad)(yp, rows_p, cols_p)
    return out[:n]


def _spread_pad(idx, epad, n_rows):
    e = idx.shape[0]
    pad = jnp.arange(epad - e, dtype=jnp.int32) % jnp.int32(n_rows)
    return jnp.concatenate([idx.astype(jnp.int32), pad])


def _gather_rows(table, idx):
    e = idx.shape[0]
    epad = _pad_len(e)
    idx_p = _spread_pad(idx, epad, table.shape[0])
    return _sc_gather(table.shape[1], epad)(table, idx_p)


def kernel(attribute_triples, ent_edges, ent_edge_labels, val_edges,
           val_edge_labels, att_feats, val_feats, ent_feats, W, gcn1_W,
           gcn1_b, gcn2_W, gcn2_b, gat1_W, gat1_att_src, gat1_att_dst,
           gat1_bias, gat1re_W, gat1re_att_src, gat1re_att_dst, gat1re_bias):
    val = attribute_triples[:, 1]
    att = attribute_triples[:, 2]
    num_ent = ent_feats.shape[0]
    key_dim = ent_feats.shape[1]
    n = num_ent + val.shape[0]

    att_proj = att_feats @ W[:key_dim]
    val_proj = val_feats @ W[key_dim:]
    vfeat = _gather_rows(att_proj, att)[: val.shape[0]] \
        + _gather_rows(val_proj, val)[: val.shape[0]]
    x0 = jnp.concatenate([ent_feats, vfeat], axis=0)

    ve_row = val_edges[:, 0]
    ve_col = val_edges[:, 1]
    e_val = ve_row.shape[0]
    epad_v = _pad_len(e_val)
    ve_col_p = jnp.pad(ve_col, (0, epad_v - e_val), constant_values=n)

    deg = jax.ops.segment_sum(jnp.ones(e_val, jnp.float32), ve_col,
                              num_segments=n) + 1.0
    dinv = deg ** -0.5
    dinv2 = dinv * dinv

    def gcn_agg(x):
        y = x * dinv[:, None]
        return _gcn_agg(y, ve_row, ve_col, n) * dinv[:, None]

    x1 = _mm_bias(gcn_agg(x0), gcn1_W, gcn1_b)
    x2 = _mm_bias(gcn_agg(x1), gcn2_W, gcn2_b)

    ee_row = ent_edges[:, 0]
    ee_col = ent_edges[:, 1]
    e_ent = ee_row.shape[0]
    epad_e = _pad_len(e_ent)
    ee_col_p = jnp.pad(ee_col, (0, epad_e - e_ent), constant_values=n)

    u1 = gat1_W @ gat1_att_src
    v1 = gat1_W @ gat1_att_dst
    u2 = gat1re_W @ gat1re_att_src
    v2 = gat1re_W @ gat1re_att_dst
    proj = x2 @ jnp.stack([u1, v1, u2, v2], axis=1)
    s1 = _gather_scalars(proj[:, 0], ee_row)
    d1 = _gather_scalars(proj[:, 1], ee_col)
    s2 = _gather_scalars(proj[:, 2], ee_row)
    d2 = _gather_scalars(proj[:, 3], ee_col)

    def leaky(a):
        return jnp.where(a > 0, a, 0.2 * a)

    ex1 = jnp.exp(leaky(s1 + d1))
    ex2 = jnp.exp(leaky(s2 + d2))
    exs1 = jnp.exp(leaky(proj[:, 0] + proj[:, 1]))
    exs2 = jnp.exp(leaky(proj[:, 2] + proj[:, 3]))

    aggu1 = _gat_agg(x2, x2 * exs1[:, None], ee_row, ee_col, ex1, n)
    aggu2 = _gat_agg(x2, x2 * exs2[:, None], ee_row, ee_col, ex2, n)
    den1 = jax.ops.segment_sum(ex1, ee_col, num_segments=n) + exs1
    den2 = jax.ops.segment_sum(ex2, ee_col, num_segments=n) + exs2
    aggu1 = aggu1 / (den1 + 1e-16)[:, None]
    aggu2 = aggu2 / (den2 + 1e-16)[:, None]

    cat = jnp.concatenate([aggu1, aggu2], axis=1)
    wcat = jnp.concatenate([gat1_W, gat1re_W], axis=0)
    out = _mm_bias(cat, wcat, gat1_bias + gat1re_bias)
    return out[:num_ent] + ent_feats


# speedup vs baseline: 14.6068x; 1.1973x over previous
import functools

import jax
import jax.numpy as jnp
from jax import lax
from jax.experimental import pallas as pl
from jax.experimental.pallas import tpu as pltpu
from jax.experimental.pallas import tpu_sc as plsc

_SC_INFO = plsc.get_sparse_core_info()
_NC = _SC_INFO.num_cores
_NS = _SC_INFO.num_subcores
_NW = _NC * _NS
_BT = 512


def _mm_bias(x, w, b, block=2000):
    n, k = x.shape
    m = w.shape[1]
    assert n % block == 0

    def body(x_ref, w_ref, b_ref, o_ref):
        o_ref[...] = (
            jnp.dot(x_ref[...], w_ref[...], preferred_element_type=jnp.float32)
            + b_ref[...]
        )

    return pl.pallas_call(
        body,
        grid=(n // block,),
        in_specs=[
            pl.BlockSpec((block, k), lambda i: (i, 0)),
            pl.BlockSpec((k, m), lambda i: (0, 0)),
            pl.BlockSpec((1, m), lambda i: (0, 0)),
        ],
        out_specs=pl.BlockSpec((block, m), lambda i: (i, 0)),
        out_shape=jax.ShapeDtypeStruct((n, m), jnp.float32),
    )(x, w, b.reshape(1, m))


@functools.cache
def _sc_gather(d, epad):
    nb = epad // (_NW * _BT)
    mesh = plsc.VectorSubcoreMesh(core_axis_name="c", subcore_axis_name="s")

    @functools.partial(
        pl.kernel,
        mesh=mesh,
        out_type=jax.ShapeDtypeStruct((epad, d), jnp.float32),
        scratch_types=[
            pltpu.VMEM((_BT,), jnp.int32),
            pltpu.VMEM((_BT, d), jnp.float32),
            pltpu.SemaphoreType.DMA,
        ],
    )
    def k(table_hbm, idx_hbm, out_hbm, idx_v, rows_v, sem):
        wid = lax.axis_index("s") * _NC + lax.axis_index("c")

        def batch(i, carry):
            base = (wid * nb + i) * _BT
            pltpu.sync_copy(idx_hbm.at[pl.ds(base, _BT)], idx_v)
            pltpu.async_copy(table_hbm.at[idx_v], rows_v, sem).wait()
            pltpu.sync_copy(rows_v, out_hbm.at[pl.ds(base, _BT)])
            return carry

        lax.fori_loop(0, nb, batch, 0)

    return k


_BTS = 2048


@functools.cache
def _sc_scalar_gather(n_table, epad, dtype=jnp.float32):
    nb = epad // (_NW * _BTS)
    mesh = plsc.VectorSubcoreMesh(core_axis_name="c", subcore_axis_name="s")

    @functools.partial(
        pl.kernel,
        mesh=mesh,
        out_type=jax.ShapeDtypeStruct((epad,), dtype),
        scratch_types=[
            pltpu.VMEM((n_table,), dtype),
            pltpu.VMEM((_BTS,), jnp.int32),
            pltpu.VMEM((_BTS,), dtype),
        ],
        compiler_params=pltpu.CompilerParams(needs_layout_passes=False),
    )
    def k(table_hbm, idx_hbm, out_hbm, tab_v, idx_v, out_v):
        wid = lax.axis_index("s") * _NC + lax.axis_index("c")
        pltpu.sync_copy(table_hbm, tab_v)

        def batch(i, carry):
            base = (wid * nb + i) * _BTS
            pltpu.sync_copy(idx_hbm.at[pl.ds(base, _BTS)], idx_v)

            def vec(j, c2):
                ii = idx_v[pl.ds(j * 16, 16)]
                out_v[pl.ds(j * 16, 16)] = plsc.load_gather(tab_v, [ii])
                return c2

            lax.fori_loop(0, _BTS // 16, vec, 0)
            pltpu.sync_copy(out_v, out_hbm.at[pl.ds(base, _BTS)])
            return carry

        lax.fori_loop(0, nb, batch, 0)

    return k


def _gather_scalars(table, idx):
    e = idx.shape[0]
    step = _NW * _BTS
    epad = ((e + step - 1) // step) * step
    idx_p = jnp.pad(idx.astype(jnp.int32), (0, epad - e))
    return _sc_scalar_gather(table.shape[0], epad, table.dtype.type)(
        table, idx_p)[:e]


def _gather_scalars_i32(table, idx):
    return _gather_scalars(table.astype(jnp.int32), idx)


def _pad_len(e):
    step = _NW * _BT
    return ((e + step - 1) // step) * step


_NHIST = 100352


@functools.cache
def _sc_histogram(epad):
    nb = epad // (_NW * _BTS)
    slc = _NHIST // 16
    mesh = plsc.VectorSubcoreMesh(core_axis_name="c", subcore_axis_name="s")

    @functools.partial(
        pl.kernel,
        mesh=mesh,
        out_type=(
            jax.ShapeDtypeStruct((2, 16, _NHIST), jnp.float32),
            jax.ShapeDtypeStruct((2, _NHIST), jnp.float32),
        ),
        scratch_types=[
            pltpu.VMEM((_NHIST,), jnp.float32),
            pltpu.VMEM((_BTS,), jnp.int32),
            pltpu.VMEM((_BTS,), jnp.float32),
            pltpu.VMEM((slc,), jnp.float32),
        ],
        compiler_params=pltpu.CompilerParams(needs_layout_passes=False),
    )
    def k(idx_hbm, w_hbm, part_hbm, out_hbm, local, idx_v, w_v, red_v):
        c = lax.axis_index("c")
        t = lax.axis_index("s")
        wid = t * _NC + c
        zero16 = jnp.zeros((16,), jnp.float32)

        def z(i, carry):
            local[pl.ds(i * 16, 16)] = zero16
            return carry

        lax.fori_loop(0, _NHIST // 16, z, 0)

        def batch(i, carry):
            base = (wid * nb + i) * _BTS
            pltpu.sync_copy(idx_hbm.at[pl.ds(base, _BTS)], idx_v)
            pltpu.sync_copy(w_hbm.at[pl.ds(base, _BTS)], w_v)

            def vec(j, c2):
                ii = idx_v[pl.ds(j * 16, 16)]
                ww = w_v[pl.ds(j * 16, 16)]
                plsc.addupdate_scatter(local, [ii], ww)
                return c2

            lax.fori_loop(0, _BTS // 16, vec, 0)
            return carry

        lax.fori_loop(0, nb, batch, 0)
        pltpu.sync_copy(local, part_hbm.at[c].at[t])
        plsc.subcore_barrier()

        def zr(i, carry):
            red_v[pl.ds(i * 16, 16)] = zero16
            return carry

        lax.fori_loop(0, slc // 16, zr, 0)
        for q in range(16):
            pltpu.sync_copy(part_hbm.at[c].at[q].at[pl.ds(t * slc, slc)],
                            local.at[pl.ds(0, slc)])

            def addq(i, carry):
                red_v[pl.ds(i * 16, 16)] = (
                    red_v[pl.ds(i * 16, 16)] + local[pl.ds(i * 16, 16)])
                return carry

            lax.fori_loop(0, slc // 16, addq, 0)
        pltpu.sync_copy(red_v, out_hbm.at[c].at[pl.ds(t * slc, slc)])

    return k


def _histogram(idx, w, n):
    e = idx.shape[0]
    step = _NW * _BTS
    epad = ((e + step - 1) // step) * step
    idx_p = jnp.pad(idx.astype(jnp.int32), (0, epad - e))
    w_p = jnp.pad(w, (0, epad - e))
    _, out = _sc_histogram(epad)(idx_p, w_p)
    return out[0, :n] + out[1, :n]


_CHK = 13568
_DUM = 128
_NCH = 8
_STRIPE = _CHK // 16


@functools.cache
def _sc_gcn_agg(n_pad, epad):
    nblk = epad // _BT
    npass = (_NCH + 1) // 2
    n_out = _NCH * _CHK
    assert nblk % 16 == 0
    mesh = plsc.VectorSubcoreMesh(core_axis_name="c", subcore_axis_name="s")

    @functools.partial(
        pl.kernel,
        mesh=mesh,
        out_type=jax.ShapeDtypeStruct((n_out, 128), jnp.float32),
        scratch_types=[
            pltpu.VMEM_SHARED((_CHK + _DUM, 128), jnp.float32),
            pltpu.VMEM((_BT,), jnp.int32),
            pltpu.VMEM((_BT,), jnp.int32),
            pltpu.VMEM((1056,), jnp.int32),
            pltpu.VMEM((1056,), jnp.int32),
            pltpu.VMEM((4, 128), jnp.int32),
            pltpu.VMEM((128, 128), jnp.float32),
            pltpu.SemaphoreType.DMA,
        ],
        compiler_params=pltpu.CompilerParams(needs_layout_passes=False),
    )
    def k(y_hbm, rows_hbm, cols_hbm, out_hbm, acc, rblk, cblk, fr, fs,
          s2d, rows_v, sem):
        c = lax.axis_index("c")
        s = lax.axis_index("s")
        t = s
        lane0 = lax.iota(jnp.int32, 16)
        for m in range(66):
            fr[pl.ds(m * 16, 16)] = lane0 + m * 16

        def seg_fire(j, rbase):
            for m in range(8):
                s2d[j, pl.ds(m * 16, 16)] = fs[pl.ds(rbase + m * 16, 16)]
            pltpu.async_copy(
                y_hbm.at[fr.at[pl.ds(rbase, 128)]], rows_v, sem).wait()
            pltpu.sync_copy(rows_v, acc.at[s2d.at[j]], add=True)

        def one_pass(j, carry):
            ch = c + 2 * j
            base = ch * _CHK
            live = ch < _NCH

            @pl.when(live)
            def _():
                pltpu.sync_copy(
                    y_hbm.at[pl.ds(base + t * _STRIPE, _STRIPE)],
                    acc.at[pl.ds(t * _STRIPE, _STRIPE)])
            plsc.subcore_barrier()

            def block(b, f):
                blk = (b * 16 + t) * _BT
                pltpu.sync_copy(rows_hbm.at[pl.ds(blk, _BT)], rblk)
                pltpu.sync_copy(cols_hbm.at[pl.ds(blk, _BT)], cblk)

                def compress_one(kk, f):
                    c16 = cblk[pl.ds(kk * 16, 16)]
                    li = c16 - base
                    msk = (li >= 0) & (li < _CHK)
                    r16 = rblk[pl.ds(kk * 16, 16)]
                    plsc.store_compressed(fr.at[pl.ds(f, 16)], r16, mask=msk)
                    plsc.store_compressed(fs.at[pl.ds(f, 16)], li, mask=msk)
                    return f + jnp.max(plsc.all_reduce_population_count(msk))

                f = lax.fori_loop(0, _BT // 16, compress_one, f)
                fired = f >= 512

                @pl.when(fired)
                def _():
                    for j4 in range(4):
                        seg_fire(j4, j4 * 128)
                    for m in range(32):
                        fr[pl.ds(m * 16, 16)] = fr[pl.ds(512 + m * 16, 16)]
                        fs[pl.ds(m * 16, 16)] = fs[pl.ds(512 + m * 16, 16)]

                return jnp.where(fired, f - 512, f)

            f = lax.fori_loop(0, nblk // 16, block, jnp.int32(0))

            lane = lax.iota(jnp.int32, 16)
            for m in range(8):
                fs[pl.ds(f + m * 16, 16)] = _CHK + ((lane + m * 16) & (_DUM - 1))

            def flush_seg(j2, carry2):
                seg_fire(j2, j2 * 128)
                return carry2

            lax.fori_loop(0, (f + 127) >> 7, flush_seg, 0)
            plsc.subcore_barrier()

            @pl.when(live)
            def _():
                pltpu.sync_copy(
                    acc.at[pl.ds(t * _STRIPE, _STRIPE)],
                    out_hbm.at[pl.ds(base + t * _STRIPE, _STRIPE)])
            plsc.subcore_barrier()
            return carry

        lax.fori_loop(0, npass, one_pass, 0)

    return k


_CHK2 = 11264
_NCH2 = 9
_STRIPE2 = _CHK2 // 16


@functools.cache
def _sc_gat_agg(n_pad, epad):
    nblk = epad // _BT
    npass = (_NCH2 + 1) // 2
    n_out = _NCH2 * _CHK2
    assert nblk % 16 == 0
    mesh = plsc.VectorSubcoreMesh(core_axis_name="c", subcore_axis_name="s")

    @functools.partial(
        pl.kernel,
        mesh=mesh,
        out_type=jax.ShapeDtypeStruct((n_out, 128), jnp.float32),
        scratch_types=[
            pltpu.VMEM_SHARED((_CHK2 + _DUM, 128), jnp.float32),
            pltpu.VMEM((_BT,), jnp.int32),
            pltpu.VMEM((_BT,), jnp.int32),
            pltpu.VMEM((_BT,), jnp.float32),
            pltpu.VMEM((1056,), jnp.int32),
            pltpu.VMEM((1056,), jnp.int32),
            pltpu.VMEM((1056,), jnp.float32),
            pltpu.VMEM((4, 128), jnp.int32),
            pltpu.VMEM((128, 128), jnp.float32),
            pltpu.VMEM((128, 128), jnp.float32),
            pltpu.SemaphoreType.DMA,
        ],
        compiler_params=pltpu.CompilerParams(needs_layout_passes=False),
    )
    def k(y_hbm, init_hbm, rows_hbm, cols_hbm, w_hbm, out_hbm, acc, rblk,
          cblk, wblk, fr, fs, fw, s2d, rows_v, sc_v, sem):
        c = lax.axis_index("c")
        t = lax.axis_index("s")
        lane0 = lax.iota(jnp.int32, 16)
        for m in range(66):
            fr[pl.ds(m * 16, 16)] = lane0 + m * 16
            fw[pl.ds(m * 16, 16)] = jnp.zeros((16,), jnp.float32)

        def seg_fire(j, rbase):
            for m in range(8):
                s2d[j, pl.ds(m * 16, 16)] = fs[pl.ds(rbase + m * 16, 16)]
            pltpu.async_copy(
                y_hbm.at[fr.at[pl.ds(rbase, 128)]], rows_v, sem).wait()

            def scale_row(r, c2):
                w = plsc.load_gather(
                    fw, [jnp.broadcast_to(rbase + r, (16,))])
                for m in range(8):
                    sc_v[r, pl.ds(m * 16, 16)] = (
                        rows_v[r, pl.ds(m * 16, 16)] * w)
                return c2

            lax.fori_loop(0, 128, scale_row, 0)
            pltpu.sync_copy(sc_v, acc.at[s2d.at[j]], add=True)

        def one_pass(j, carry):
            ch = c + 2 * j
            base = ch * _CHK2
            live = ch < _NCH2

            @pl.when(live)
            def _():
                pltpu.sync_copy(
                    init_hbm.at[pl.ds(base + t * _STRIPE2, _STRIPE2)],
                    acc.at[pl.ds(t * _STRIPE2, _STRIPE2)])
            plsc.subcore_barrier()

            def block(b, f):
                blk = (b * 16 + t) * _BT
                pltpu.sync_copy(rows_hbm.at[pl.ds(blk, _BT)], rblk)
                pltpu.sync_copy(cols_hbm.at[pl.ds(blk, _BT)], cblk)
                pltpu.sync_copy(w_hbm.at[pl.ds(blk, _BT)], wblk)

                def compress_one(kk, f):
                    c16 = cblk[pl.ds(kk * 16, 16)]
                    li = c16 - base
                    msk = (li >= 0) & (li < _CHK2)
                    r16 = rblk[pl.ds(kk * 16, 16)]
                    w16 = wblk[pl.ds(kk * 16, 16)]
                    plsc.store_compressed(fr.at[pl.ds(f, 16)], r16, mask=msk)
                    plsc.store_compressed(fs.at[pl.ds(f, 16)], li, mask=msk)
                    plsc.store_compressed(fw.at[pl.ds(f, 16)], w16, mask=msk)
                    return f + jnp.max(plsc.all_reduce_population_count(msk))

                f = lax.fori_loop(0, _BT // 16, compress_one, f)
                fired = f >= 512

                @pl.when(fired)
                def _():
                    for j4 in range(4):
                        seg_fire(j4, j4 * 128)
                    for m in range(32):
                        fr[pl.ds(m * 16, 16)] = fr[pl.ds(512 + m * 16, 16)]
                        fs[pl.ds(m * 16, 16)] = fs[pl.ds(512 + m * 16, 16)]
                        fw[pl.ds(m * 16, 16)] = fw[pl.ds(512 + m * 16, 16)]

                return jnp.where(fired, f - 512, f)

            f = lax.fori_loop(0, nblk // 16, block, jnp.int32(0))

            lane = lax.iota(jnp.int32, 16)
            for m in range(8):
                fs[pl.ds(f + m * 16, 16)] = (
                    _CHK2 + ((lane + m * 16) & (_DUM - 1)))

            def flush_seg(j2, carry2):
                seg_fire(j2, j2 * 128)
                return carry2

            lax.fori_loop(0, (f + 127) >> 7, flush_seg, 0)
            plsc.subcore_barrier()

            @pl.when(live)
            def _():
                pltpu.sync_copy(
                    acc.at[pl.ds(t * _STRIPE2, _STRIPE2)],
                    out_hbm.at[pl.ds(base + t * _STRIPE2, _STRIPE2)])
            plsc.subcore_barrier()
            return carry

        lax.fori_loop(0, npass, one_pass, 0)

    return k


def _gat_agg(y, init, rows, cols, w, n):
    e = rows.shape[0]
    epad = _pad_len(e)
    n_out = _NCH2 * _CHK2
    init_p = jnp.pad(init, ((0, n_out - n), (0, 0)))
    rows_p = _spread_pad(rows, epad, n)
    cols_p = jnp.pad(cols.astype(jnp.int32), (0, epad - e),
                     constant_values=n_out + _CHK2)
    w_p = jnp.pad(w, (0, epad - e))
    out = _sc_gat_agg(n_out, epad)(y, init_p, rows_p, cols_p, w_p)
    return out[:n]


def _gcn_agg(y_init, rows, cols, n):
    e = rows.shape[0]
    epad = _pad_len(e)
    n_out = _NCH * _CHK
    yp = jnp.pad(y_init, ((0, n_out - n), (0, 0)))
    rows_p = _spread_pad(rows, epad, n)
    cols_p = jnp.pad(cols.astype(jnp.int32), (0, epad - e),
                     constant_values=n_out + _CHK)
    out = _sc_gcn_agg(n_out, epad)(yp, rows_p, cols_p)
    return out[:n]


def _spread_pad(idx, epad, n_rows):
    e = idx.shape[0]
    pad = jnp.arange(epad - e, dtype=jnp.int32) % jnp.int32(n_rows)
    return jnp.concatenate([idx.astype(jnp.int32), pad])


def _gather_rows(table, idx):
    e = idx.shape[0]
    epad = _pad_len(e)
    idx_p = _spread_pad(idx, epad, table.shape[0])
    return _sc_gather(table.shape[1], epad)(table, idx_p)


def kernel(attribute_triples, ent_edges, ent_edge_labels, val_edges,
           val_edge_labels, att_feats, val_feats, ent_feats, W, gcn1_W,
           gcn1_b, gcn2_W, gcn2_b, gat1_W, gat1_att_src, gat1_att_dst,
           gat1_bias, gat1re_W, gat1re_att_src, gat1re_att_dst, gat1re_bias):
    val = attribute_triples[:, 1]
    att = attribute_triples[:, 2]
    num_ent = ent_feats.shape[0]
    key_dim = ent_feats.shape[1]
    n = num_ent + val.shape[0]

    att_proj = att_feats @ W[:key_dim]
    val_proj = val_feats @ W[key_dim:]
    vfeat = _gather_rows(att_proj, att)[: val.shape[0]] \
        + _gather_rows(val_proj, val)[: val.shape[0]]
    x0 = jnp.concatenate([ent_feats, vfeat], axis=0)

    ve_row = val_edges[:, 0]
    ve_col = val_edges[:, 1]
    e_val = ve_row.shape[0]
    deg = _histogram(ve_col, jnp.ones(e_val, jnp.float32), n) + 1.0
    dinv = deg ** -0.5
    dinv2 = dinv * dinv

    def gcn_agg(x):
        y = x * dinv[:, None]
        return _gcn_agg(y, ve_row, ve_col, n) * dinv[:, None]

    x1 = _mm_bias(gcn_agg(x0), gcn1_W, gcn1_b)
    x2 = _mm_bias(gcn_agg(x1), gcn2_W, gcn2_b)

    ee_row = ent_edges[:, 0]
    ee_col = ent_edges[:, 1]

    u1 = gat1_W @ gat1_att_src
    v1 = gat1_W @ gat1_att_dst
    u2 = gat1re_W @ gat1re_att_src
    v2 = gat1re_W @ gat1re_att_dst
    proj = x2 @ jnp.stack([u1, v1, u2, v2], axis=1)
    s1 = _gather_scalars(proj[:, 0], ee_row)
    d1 = _gather_scalars(proj[:, 1], ee_col)
    s2 = _gather_scalars(proj[:, 2], ee_row)
    d2 = _gather_scalars(proj[:, 3], ee_col)

    def leaky(a):
        return jnp.where(a > 0, a, 0.2 * a)

    ex1 = jnp.exp(leaky(s1 + d1))
    ex2 = jnp.exp(leaky(s2 + d2))
    exs1 = jnp.exp(leaky(proj[:, 0] + proj[:, 1]))
    exs2 = jnp.exp(leaky(proj[:, 2] + proj[:, 3]))

    aggu1 = _gat_agg(x2, x2 * exs1[:, None], ee_row, ee_col, ex1, n)
    aggu2 = _gat_agg(x2, x2 * exs2[:, None], ee_row, ee_col, ex2, n)
    den1 = _histogram(ee_col, ex1, n) + exs1
    den2 = _histogram(ee_col, ex2, n) + exs2
    aggu1 = aggu1 / (den1 + 1e-16)[:, None]
    aggu2 = aggu2 / (den2 + 1e-16)[:, None]

    cat = jnp.concatenate([aggu1, aggu2], axis=1)
    wcat = jnp.concatenate([gat1_W, gat1re_W], axis=0)
    out = _mm_bias(cat, wcat, gat1_bias + gat1re_bias)
    return out[:num_ent] + ent_feats
